# baseline (speedup 1.0000x reference)
import jax
import jax.numpy as jnp
from jax import lax
from jax.experimental import pallas as pl
from jax.experimental.pallas import tpu as pltpu
from jax.experimental.pallas import tpu_sc as plsc

N = 10000
E = 160000
D = 256
L = 16
DP = D + L
DPB = D + 2 * L
NP = 10240
NC = 2
NS = 16
HALF = N // NC
NPASS = 5
QTR = HALF // NPASS
EPT = E // NS
BM = 256
CH = 32
QSZ = EPT + 3 * CH

RPT = 62
RCH = 2


def _tc_body(x_ref, w_ref, a2_ref, h_ref, al_ref):
    h = jnp.dot(x_ref[...], w_ref[...], preferred_element_type=jnp.float32)
    col = lax.broadcasted_iota(jnp.int32, (BM, 2 * L), 1)
    pad = jnp.where(col == 0, 1.0, 0.0).astype(jnp.float32)
    h_ref[...] = jnp.concatenate([h, pad], axis=1).astype(jnp.bfloat16)
    al_ref[...] = jnp.dot(h, a2_ref[...], preferred_element_type=jnp.float32)


_tc = pl.pallas_call(
    _tc_body,
    grid=(NP // BM,),
    in_specs=[
        pl.BlockSpec((BM, D), lambda i: (i, 0)),
        pl.BlockSpec((D, D), lambda i: (0, 0)),
        pl.BlockSpec((D, 128), lambda i: (0, 0)),
    ],
    out_specs=[
        pl.BlockSpec((BM, DPB), lambda i: (i, 0)),
        pl.BlockSpec((BM, 128), lambda i: (i, 0)),
    ],
    out_shape=[
        jax.ShapeDtypeStruct((NP, DPB), jnp.bfloat16),
        jax.ShapeDtypeStruct((NP, 128), jnp.float32),
    ],
)


def _sc_body(h, src_e, dst_e, asrc, adst, bias, out,
             src_v, dst_v, p_all, asrc_v, adst_v,
             q_p, q_s, q_d, rows_a, rows_b, stage, obuf, obuf2,
             bias_v, acc, sem):
    c = lax.axis_index("c")
    s = lax.axis_index("s")
    lo = c * HALF
    rstart = s * RPT
    ntrip = jnp.where(s == NS - 1, 35, 31)
    zv = jnp.zeros((L,), jnp.float32)
    zi = jnp.zeros((L,), jnp.int32)

    with jax.named_scope("sc_stage"):
        ebase = s * EPT
        pltpu.sync_copy(src_e.at[pl.ds(ebase, EPT)], src_v)
        pltpu.sync_copy(dst_e.at[pl.ds(ebase, EPT)], dst_v)
        pltpu.sync_copy(asrc, asrc_v)
        pltpu.sync_copy(adst, adst_v)
        pltpu.sync_copy(bias, bias_v)

    def e_body(i, carry):
        s16 = src_v[pl.ds(i * L, L)]
        d16 = dst_v[pl.ds(i * L, L)]
        av = plsc.load_gather(asrc_v, [s16])
        bv = plsc.load_gather(adst_v, [d16])
        e = av + bv
        e = jnp.where(e > 0, e, 0.2 * e)
        p_all[pl.ds(i * L, L)] = jnp.exp(e)
        return carry

    with jax.named_scope("sc_phase1"):
        lax.fori_loop(0, EPT // L, e_body, 0)

    for i in range(RCH):
        for q in range(DP // L):
            obuf[i, pl.ds(q * L, L)] = zv

    def pass_body(t, carry):
        qlo = t * QTR

        def r_body(i, off):
            s16 = src_v[pl.ds(i * L, L)]
            d16 = dst_v[pl.ds(i * L, L)]
            pv = p_all[pl.ds(i * L, L)]
            dq = d16 - (lo + qlo)
            mk = (dq >= 0) & (dq < QTR)
            plsc.store_compressed(q_p.at[pl.ds(off, L)], pv, mask=mk)
            plsc.store_compressed(q_s.at[pl.ds(off, L)], s16, mask=mk)
            plsc.store_compressed(q_d.at[pl.ds(off, L)], dq, mask=mk)
            return off + jnp.sum(mk.astype(jnp.int32))

        with jax.named_scope("sc_compact"):
            mq = lax.fori_loop(0, EPT // L, r_body, jnp.int32(0))

            def pad_body(k, carry):
                q_p[pl.ds(mq + k * L, L)] = zv
                q_s[pl.ds(mq + k * L, L)] = zi
                q_d[pl.ds(mq + k * L, L)] = zi
                return carry

            lax.fori_loop(0, 3 * CH // L, pad_body, 0)

        def z_body(k, carry):
            pltpu.sync_copy(obuf, acc.at[pl.ds(rstart + k * RCH, RCH)])
            return carry

        with jax.named_scope("sc_zero"):
            lax.fori_loop(0, ntrip, z_body, 0)
        plsc.subcore_barrier()

        nchb = (mq + (CH - 1)) // CH
        nchb2 = (nchb + 1) // 2

        def scale_scatter(rbuf, j):
            for b in range(CH // L):
                base = j * CH + b * L
                pv = q_p[pl.ds(base, L)]
                dv = q_d[pl.ds(base, L)]
                for i in range(L):
                    piv = jnp.full((L,), pv[i], jnp.float32)
                    for g in range(DPB // (2 * L)):
                        vi = plsc.bitcast(
                            rbuf[b * L + i, pl.ds(g * 2 * L, 2 * L)],
                            jnp.int32)
                        flo = plsc.bitcast(vi << 16, jnp.float32)
                        stage[i, pl.ds(g * 2 * L, L)] = flo * piv
                        if g * 2 * L + L < DP:
                            fhi = plsc.bitcast(
                                vi & jnp.int32(-65536), jnp.float32)
                            stage[i, pl.ds(g * 2 * L + L, L)] = fhi * piv
                pltpu.sync_copy(stage, acc.at[dv], add=True)

        pltpu.async_copy(h.at[q_s.at[pl.ds(0, CH)]], rows_a, sem)

        def g_body(jj, carry):
            j0 = 2 * jj
            pltpu.async_copy(
                h.at[q_s.at[pl.ds((j0 + 1) * CH, CH)]], rows_b, sem).wait()
            scale_scatter(rows_a, j0)
            pltpu.async_copy(
                h.at[q_s.at[pl.ds((j0 + 2) * CH, CH)]], rows_a, sem).wait()
            scale_scatter(rows_b, j0 + 1)
            return carry

        with jax.named_scope("sc_gather_scatter"):
            lax.fori_loop(0, nchb2, g_body, 0)
            pltpu.make_async_copy(h.at[pl.ds(0, CH)], rows_a, sem).wait()
        plsc.subcore_barrier()

        def w_body(k, carry):
            base = rstart + k * RCH
            pltpu.sync_copy(acc.at[pl.ds(base, RCH)],
                            stage.at[pl.ds(0, RCH)])
            for i in range(RCH):
                dv16 = stage[i, pl.ds(D, L)]
                den = jnp.full((L,), dv16[0], jnp.float32) + 1e-16
                r = 1.0 / den
                for q in range(D // L):
                    obuf2[i, pl.ds(q * L, L)] = (
                        stage[i, pl.ds(q * L, L)] * r
                        + bias_v[pl.ds(q * L, L)]
                    )
            pltpu.sync_copy(obuf2, out.at[pl.ds(lo + qlo + base, RCH)])
            return carry

        with jax.named_scope("sc_writeout"):
            lax.fori_loop(0, ntrip, w_body, 0)
        plsc.subcore_barrier()
        return carry

    lax.fori_loop(0, NPASS, pass_body, 0)


_sc = pl.kernel(
    _sc_body,
    out_type=jax.ShapeDtypeStruct((N, D), jnp.float32),
    mesh=plsc.VectorSubcoreMesh(core_axis_name="c", subcore_axis_name="s"),
    compiler_params=pltpu.CompilerParams(
        needs_layout_passes=False, use_tc_tiling_on_sc=False
    ),
    scratch_types=[
        pltpu.VMEM((EPT,), jnp.int32),
        pltpu.VMEM((EPT,), jnp.int32),
        pltpu.VMEM((EPT,), jnp.float32),
        pltpu.VMEM((NP,), jnp.float32),
        pltpu.VMEM((NP,), jnp.float32),
        pltpu.VMEM((QSZ,), jnp.float32),
        pltpu.VMEM((QSZ,), jnp.int32),
        pltpu.VMEM((QSZ,), jnp.int32),
        pltpu.VMEM((CH, DPB), jnp.bfloat16),
        pltpu.VMEM((CH, DPB), jnp.bfloat16),
        pltpu.VMEM((L, DP), jnp.float32),
        pltpu.VMEM((RCH, DP), jnp.float32),
        pltpu.VMEM((RCH, D), jnp.float32),
        pltpu.VMEM((D,), jnp.float32),
        pltpu.VMEM_SHARED((QTR, DP), jnp.float32),
        pltpu.SemaphoreType.DMA,
    ],
)


def _layer(xp, src, dst, W, a_src, a_dst, b):
    a2 = jnp.pad(jnp.stack([a_src, a_dst], axis=1), ((0, 0), (0, 126)))
    ha, al = _tc(xp, W, a2)
    hi = ha.reshape(NP, DPB // (2 * L), 2, L).swapaxes(2, 3).reshape(NP, DPB)
    return _sc(hi, src, dst, al[:, 0], al[:, 1], b)


def kernel(x, edge_index, W1, a_src1, a_dst1, b1, W2, a_src2, a_dst2, b2):
    src = edge_index[0].astype(jnp.int32)
    dst = edge_index[1].astype(jnp.int32)
    xp = jnp.pad(x, ((0, NP - N), (0, 0)))
    x1 = _layer(xp, src, dst, W1, a_src1, a_dst1, b1)
    xp1 = jnp.pad(x1, ((0, NP - N), (0, 0)))
    return _layer(xp1, src, dst, W2, a_src2, a_dst2, b2)


# speedup vs baseline: 1.4138x; 1.4138x over previous
import jax
import jax.numpy as jnp
from jax import lax
from jax.experimental import pallas as pl
from jax.experimental.pallas import tpu as pltpu
from jax.experimental.pallas import tpu_sc as plsc

N = 10000
E = 160000
D = 256
L = 16
DP = D + L
NP = 10240
NC = 2
NS = 16
HALF = N // NC
QTR = HALF // 4
EPT = E // NS
BM = 256

RPT = 78
RCH = 2


def _tc_body(x_ref, w_ref, a2_ref, h_ref, al_ref):
    h = jnp.dot(x_ref[...], w_ref[...], preferred_element_type=jnp.float32)
    col = lax.broadcasted_iota(jnp.int32, (BM, L), 1)
    pad = jnp.where(col == 0, 1.0, 0.0).astype(jnp.float32)
    h_ref[...] = jnp.concatenate([h, pad], axis=1)
    al_ref[...] = jnp.dot(h, a2_ref[...], preferred_element_type=jnp.float32)


_tc = pl.pallas_call(
    _tc_body,
    grid=(NP // BM,),
    in_specs=[
        pl.BlockSpec((BM, D), lambda i: (i, 0)),
        pl.BlockSpec((D, D), lambda i: (0, 0)),
        pl.BlockSpec((D, 128), lambda i: (0, 0)),
    ],
    out_specs=[
        pl.BlockSpec((BM, DP), lambda i: (i, 0)),
        pl.BlockSpec((BM, 128), lambda i: (i, 0)),
    ],
    out_shape=[
        jax.ShapeDtypeStruct((NP, DP), jnp.float32),
        jax.ShapeDtypeStruct((NP, 128), jnp.float32),
    ],
)


def _sc_body(h, src_e, dst_e, asrc, adst, bias, out,
             src_v, dst_v, asrc_v, adst_v,
             p_c0, s_c0, d_c0, p_c1, s_c1, d_c1,
             rows, obuf, obuf2, bias_v, acc, sem):
    c = lax.axis_index("c")
    s = lax.axis_index("s")
    lo = c * HALF
    rstart = s * RPT
    ntrip = jnp.where(s == NS - 1, 40, 39)

    ebase = s * EPT
    pltpu.sync_copy(src_e.at[pl.ds(ebase, EPT)], src_v.at[pl.ds(0, EPT)])
    pltpu.sync_copy(dst_e.at[pl.ds(ebase, EPT)], dst_v.at[pl.ds(0, EPT)])
    pltpu.sync_copy(asrc, asrc_v)
    pltpu.sync_copy(adst, adst_v)
    pltpu.sync_copy(bias, bias_v)

    def e_body(i, offs):
        off0, off1 = offs
        s16 = src_v[pl.ds(i * L, L)]
        d16 = dst_v[pl.ds(i * L, L)]
        av = plsc.load_gather(asrc_v, [s16])
        bv = plsc.load_gather(adst_v, [d16])
        e = av + bv
        e = jnp.where(e > 0, e, 0.2 * e)
        p = jnp.exp(e)
        dloc = d16 - lo
        m0 = (dloc >= 0) & (dloc < 2 * QTR)
        m1 = (dloc >= 2 * QTR) & (dloc < HALF)
        plsc.store_compressed(p_c0.at[pl.ds(off0, L)], p, mask=m0)
        plsc.store_compressed(s_c0.at[pl.ds(off0, L)], s16, mask=m0)
        plsc.store_compressed(d_c0.at[pl.ds(off0, L)], dloc, mask=m0)
        plsc.store_compressed(p_c1.at[pl.ds(off1, L)], p, mask=m1)
        plsc.store_compressed(s_c1.at[pl.ds(off1, L)], s16, mask=m1)
        plsc.store_compressed(d_c1.at[pl.ds(off1, L)], dloc, mask=m1)
        return (off0 + jnp.sum(m0.astype(jnp.int32)),
                off1 + jnp.sum(m1.astype(jnp.int32)))

    m0, m1 = lax.fori_loop(0, EPT // L, e_body,
                           (jnp.int32(0), jnp.int32(0)))

    zv = jnp.zeros((L,), jnp.float32)
    zi = jnp.zeros((L,), jnp.int32)
    ni = jnp.full((L,), -1, jnp.int32)
    p_c0[pl.ds(m0, L)] = zv
    s_c0[pl.ds(m0, L)] = zi
    d_c0[pl.ds(m0, L)] = ni
    p_c1[pl.ds(m1, L)] = zv
    s_c1[pl.ds(m1, L)] = zi
    d_c1[pl.ds(m1, L)] = ni

    q_p, q_s, q_d = asrc_v, src_v, dst_v

    for t in range(4):
        p_c, s_c, d_c, m = ((p_c0, s_c0, d_c0, m0) if t < 2
                            else (p_c1, s_c1, d_c1, m1))
        qlo = t * QTR

        def r_body(i, off):
            pv = p_c[pl.ds(i * L, L)]
            sv = s_c[pl.ds(i * L, L)]
            dv = d_c[pl.ds(i * L, L)]
            dq = dv - qlo
            mk = (dq >= 0) & (dq < QTR)
            plsc.store_compressed(q_p.at[pl.ds(off, L)], pv, mask=mk)
            plsc.store_compressed(q_s.at[pl.ds(off, L)], sv, mask=mk)
            plsc.store_compressed(q_d.at[pl.ds(off, L)], dq, mask=mk)
            return off + jnp.sum(mk.astype(jnp.int32))

        nparent = (m + (L - 1)) // L
        mq = lax.fori_loop(0, nparent, r_body, jnp.int32(0))
        q_p[pl.ds(mq, L)] = zv
        q_s[pl.ds(mq, L)] = zi
        q_d[pl.ds(mq, L)] = zi

        for i in range(RCH):
            for q in range(DP // L):
                obuf[i, pl.ds(q * L, L)] = zv

        def z_body(k, carry):
            pltpu.sync_copy(obuf, acc.at[pl.ds(rstart + k * RCH, RCH)])
            return carry

        lax.fori_loop(0, ntrip, z_body, 0)
        plsc.subcore_barrier()

        nch = (mq + (L - 1)) // L

        def g_body(j, carry):
            sv = q_s[pl.ds(j * L, L)]
            dv = q_d[pl.ds(j * L, L)]
            pv = q_p[pl.ds(j * L, L)]
            pltpu.async_copy(h.at[sv], rows, sem).wait()
            for i in range(L):
                piv = jnp.full((L,), pv[i], jnp.float32)
                for q in range(DP // L):
                    rows[i, pl.ds(q * L, L)] = rows[i, pl.ds(q * L, L)] * piv
            pltpu.sync_copy(rows, acc.at[dv], add=True)
            return carry

        lax.fori_loop(0, nch, g_body, 0)
        plsc.subcore_barrier()

        def w_body(k, carry):
            base = rstart + k * RCH
            pltpu.sync_copy(acc.at[pl.ds(base, RCH)], obuf)
            for i in range(RCH):
                dv16 = obuf[i, pl.ds(D, L)]
                den = jnp.full((L,), dv16[0], jnp.float32) + 1e-16
                r = 1.0 / den
                for q in range(D // L):
                    obuf2[i, pl.ds(q * L, L)] = (
                        obuf[i, pl.ds(q * L, L)] * r + bias_v[pl.ds(q * L, L)]
                    )
            pltpu.sync_copy(obuf2, out.at[pl.ds(lo + qlo + base, RCH)])
            return carry

        lax.fori_loop(0, ntrip, w_body, 0)
        if t < 3:
            plsc.subcore_barrier()


_sc = pl.kernel(
    _sc_body,
    out_type=jax.ShapeDtypeStruct((N, D), jnp.float32),
    mesh=plsc.VectorSubcoreMesh(core_axis_name="c", subcore_axis_name="s"),
    compiler_params=pltpu.CompilerParams(
        needs_layout_passes=False, use_tc_tiling_on_sc=False
    ),
    scratch_types=[
        pltpu.VMEM((EPT + L,), jnp.int32),
        pltpu.VMEM((EPT + L,), jnp.int32),
        pltpu.VMEM((NP,), jnp.float32),
        pltpu.VMEM((NP,), jnp.float32),
        pltpu.VMEM((EPT + L,), jnp.float32),
        pltpu.VMEM((EPT + L,), jnp.int32),
        pltpu.VMEM((EPT + L,), jnp.int32),
        pltpu.VMEM((EPT + L,), jnp.float32),
        pltpu.VMEM((EPT + L,), jnp.int32),
        pltpu.VMEM((EPT + L,), jnp.int32),
        pltpu.VMEM((L, DP), jnp.float32),
        pltpu.VMEM((RCH, DP), jnp.float32),
        pltpu.VMEM((RCH, D), jnp.float32),
        pltpu.VMEM((D,), jnp.float32),
        pltpu.VMEM_SHARED((QTR, DP), jnp.float32),
        pltpu.SemaphoreType.DMA,
    ],
)


def _layer(xp, src, dst, W, a_src, a_dst, b):
    a2 = jnp.pad(jnp.stack([a_src, a_dst], axis=1), ((0, 0), (0, 126)))
    ha, al = _tc(xp, W, a2)
    return _sc(ha, src, dst, al[:, 0], al[:, 1], b)


def kernel(x, edge_index, W1, a_src1, a_dst1, b1, W2, a_src2, a_dst2, b2):
    src = edge_index[0].astype(jnp.int32)
    dst = edge_index[1].astype(jnp.int32)
    xp = jnp.pad(x, ((0, NP - N), (0, 0)))
    x1 = _layer(xp, src, dst, W1, a_src1, a_dst1, b1)
    xp1 = jnp.pad(x1, ((0, NP - N), (0, 0)))
    return _layer(xp1, src, dst, W2, a_src2, a_dst2, b2)
